# baseline (device time: 106025 ns/iter reference)
import jax
import jax.numpy as jnp
from jax import lax
from jax.experimental import pallas as pl
from jax.experimental.pallas import tpu as pltpu


def kernel(x):
    m, n = x.shape
    h = m

    def body(x_ref, out_ref, mine, comm, zsend, zrecv):
        my_x = lax.axis_index("x")
        my_y = lax.axis_index("y")
        my_z = lax.axis_index("z")
        znbr = (my_x, my_y, 1 - my_z)

        barrier_sem = pltpu.get_barrier_semaphore()
        pl.semaphore_signal(
            barrier_sem, inc=1, device_id=znbr,
            device_id_type=pl.DeviceIdType.MESH,
        )
        pl.semaphore_wait(barrier_sem, 1)

        mine[...] = x_ref[pl.ds(0, h), :].astype(jnp.bfloat16)
        zr = pltpu.make_async_remote_copy(
            src_ref=mine,
            dst_ref=comm,
            send_sem=zsend,
            recv_sem=zrecv,
            device_id=znbr,
            device_id_type=pl.DeviceIdType.MESH,
        )
        zr.start()
        zr.wait()

    return pl.pallas_call(
        body,
        out_shape=jax.ShapeDtypeStruct((2 * m, n), jnp.bfloat16),
        in_specs=[pl.BlockSpec(memory_space=pltpu.VMEM)],
        out_specs=pl.BlockSpec(memory_space=pl.ANY),
        scratch_shapes=[
            pltpu.VMEM((h, n), jnp.bfloat16),
            pltpu.VMEM((h, n), jnp.bfloat16),
            pltpu.SemaphoreType.DMA,
            pltpu.SemaphoreType.DMA,
        ],
        compiler_params=pltpu.CompilerParams(collective_id=0),
    )(x)


# device time: 58970 ns/iter; 1.7979x vs baseline; 1.7979x over previous
import jax
import jax.numpy as jnp
from jax import lax
from jax.experimental import pallas as pl
from jax.experimental.pallas import tpu as pltpu

P = 672
PH = 336
R = 704
RH = 352


def kernel(x):
    m, n = x.shape
    assert 4 * P + 2 * R == m

    def body(
        x_ref, out_ref, mine, comm,
        zsend, zrecv, xsend, xrecv, ysend, yrecv,
        lzsem, lxsem, lysem, lmsem,
    ):
        my_x = lax.axis_index("x")
        my_y = lax.axis_index("y")
        my_z = lax.axis_index("z")
        znbr = (my_x, my_y, 1 - my_z)
        xnbr = (1 - my_x, my_y, my_z)
        ynbr = (my_x, 1 - my_y, my_z)
        c = (my_x + my_y) % 2
        my_base = my_z * m
        other_base = (1 - my_z) * m

        p_own = P * (my_x + 2 * my_y)
        p_xn = P * ((1 - my_x) + 2 * my_y)
        p_yn = P * (my_x + 2 * (1 - my_y))
        p_dg = P * ((1 - my_x) + 2 * (1 - my_y))
        r_own = 4 * P + R * c
        r_oth = 4 * P + R * (1 - c)

        def rdma(src_off, dst_off, rows, ssem, rsem, nbr):
            r_ = pltpu.make_async_remote_copy(
                src_ref=(mine if src_off is None else comm).at[
                    pl.ds(dst_off if src_off is None else src_off, rows), :
                ],
                dst_ref=comm.at[pl.ds(dst_off, rows), :],
                send_sem=ssem,
                recv_sem=rsem,
                device_id=nbr,
                device_id_type=pl.DeviceIdType.MESH,
            )
            r_.start()
            return r_

        def copy_out(off, rows, sem):
            cp = pltpu.make_async_copy(
                comm.at[pl.ds(off, rows), :],
                out_ref.at[pl.ds(other_base + off, rows), :],
                sem,
            )
            cp.start()
            return cp

        barrier_sem = pltpu.get_barrier_semaphore()
        for nbr in (znbr, xnbr, ynbr):
            pl.semaphore_signal(
                barrier_sem, inc=1, device_id=nbr,
                device_id_type=pl.DeviceIdType.MESH,
            )
        pl.semaphore_wait(barrier_sem, 3)

        z_chunks = (
            (p_own, PH), (p_own + PH, PH), (r_own, RH), (r_own + RH, RH)
        )
        z_rdmas = []
        for i, (off, rows) in enumerate(z_chunks):
            mine[pl.ds(off, rows), :] = x_ref[pl.ds(off, rows), :].astype(
                jnp.bfloat16
            )
            z_rdmas.append(
                rdma(None, off, rows, zsend.at[i], zrecv.at[i], znbr)
            )

        for off, rows in (
            (p_xn, P), (p_yn, P), (p_dg, P), (r_oth, R)
        ):
            mine[pl.ds(off, rows), :] = x_ref[pl.ds(off, rows), :].astype(
                jnp.bfloat16
            )
        mcopy = pltpu.make_async_copy(
            mine, out_ref.at[pl.ds(my_base, m), :], lmsem
        )
        mcopy.start()

        sends = []
        copies = [mcopy]

        for j in range(2):
            z_rdmas[j].wait_recv()
            off = p_own + j * PH
            sends.append(rdma(off, off, PH, xsend.at[j], xrecv.at[j], xnbr))
            sends.append(rdma(off, off, PH, ysend.at[j], yrecv.at[j], ynbr))
            copies.append(copy_out(off, PH, lzsem.at[j]))

        x_in = [
            (p_xn, PH), (p_xn + PH, PH), (r_oth, RH), (p_dg, PH)
        ]
        y_in = [
            (p_yn, PH), (p_yn + PH, PH), (r_oth + RH, RH), (p_dg + PH, PH)
        ]

        def wait_in(slots, offs, rsem, lsem):
            ws = []
            for i in slots:
                off, rows = offs[i]
                w = pltpu.make_async_remote_copy(
                    src_ref=comm.at[pl.ds(off, rows), :],
                    dst_ref=comm.at[pl.ds(off, rows), :],
                    send_sem=zsend.at[0],
                    recv_sem=rsem.at[i],
                    device_id=znbr,
                    device_id_type=pl.DeviceIdType.MESH,
                )
                w.wait_recv()
                copies.append(copy_out(off, rows, lsem.at[i]))
                ws.append(w)
            return ws

        wait_in([0], x_in, xrecv, lxsem)
        wait_in([0], y_in, yrecv, lysem)
        sends.append(rdma(p_yn, p_yn, PH, xsend.at[3], xrecv.at[3], xnbr))

        z_rdmas[2].wait_recv()
        sends.append(rdma(r_own, r_own, RH, xsend.at[2], xrecv.at[2], xnbr))
        copies.append(copy_out(r_own, RH, lzsem.at[2]))

        wait_in([1], x_in, xrecv, lxsem)
        sends.append(
            rdma(p_xn + PH, p_xn + PH, PH, ysend.at[3], yrecv.at[3], ynbr)
        )

        z_rdmas[3].wait_recv()
        sends.append(
            rdma(r_own + RH, r_own + RH, RH, ysend.at[2], yrecv.at[2], ynbr)
        )
        copies.append(copy_out(r_own + RH, RH, lzsem.at[3]))

        wait_in([1, 2, 3], y_in, yrecv, lysem)
        wait_in([2, 3], x_in, xrecv, lxsem)

        for r_ in z_rdmas:
            r_.wait_send()
        for r_ in sends:
            r_.wait_send()
        for cp in copies:
            cp.wait()

    return pl.pallas_call(
        body,
        out_shape=jax.ShapeDtypeStruct((2 * m, n), jnp.bfloat16),
        in_specs=[pl.BlockSpec(memory_space=pltpu.VMEM)],
        out_specs=pl.BlockSpec(memory_space=pl.ANY),
        scratch_shapes=[
            pltpu.VMEM((m, n), jnp.bfloat16),
            pltpu.VMEM((m, n), jnp.bfloat16),
            pltpu.SemaphoreType.DMA((4,)),
            pltpu.SemaphoreType.DMA((4,)),
            pltpu.SemaphoreType.DMA((4,)),
            pltpu.SemaphoreType.DMA((4,)),
            pltpu.SemaphoreType.DMA((4,)),
            pltpu.SemaphoreType.DMA((4,)),
            pltpu.SemaphoreType.DMA((4,)),
            pltpu.SemaphoreType.DMA((4,)),
            pltpu.SemaphoreType.DMA((4,)),
            pltpu.SemaphoreType.DMA,
        ],
        compiler_params=pltpu.CompilerParams(collective_id=0),
    )(x)


# device time: 50160 ns/iter; 2.1137x vs baseline; 1.1756x over previous
import jax
import jax.numpy as jnp
from jax import lax
from jax.experimental import pallas as pl
from jax.experimental.pallas import tpu as pltpu

P = 672
PH = 336
R = 704
RH = 352

PC = (112, 224, 168, 168)
RC = (176, 176, 176, 176)


def kernel(x):
    m, n = x.shape
    assert 4 * P + 2 * R == m

    def body(
        x_ref, out_ref, stage, mine, comm,
        insem, zsend, zrecv, xsend, xrecv, ysend, yrecv,
        lzsem, lxsem, lysem, lmsem,
    ):
        my_x = lax.axis_index("x")
        my_y = lax.axis_index("y")
        my_z = lax.axis_index("z")
        znbr = (my_x, my_y, 1 - my_z)
        xnbr = (1 - my_x, my_y, my_z)
        ynbr = (my_x, 1 - my_y, my_z)
        c = (my_x + my_y) % 2
        my_base = my_z * m
        other_base = (1 - my_z) * m

        p_own = P * (my_x + 2 * my_y)
        p_xn = P * ((1 - my_x) + 2 * my_y)
        p_yn = P * (my_x + 2 * (1 - my_y))
        p_dg = P * ((1 - my_x) + 2 * (1 - my_y))
        r_own = 4 * P + R * c
        r_oth = 4 * P + R * (1 - c)

        def chunk_offs(base, sizes):
            offs, o = [], base
            for s in sizes:
                offs.append((o, s))
                o = o + s
            return offs

        z_regions = chunk_offs(p_own, PC) + chunk_offs(r_own, RC)
        rest_regions = [(p_xn, P), (p_yn, P), (p_dg, P), (r_oth, R)]

        in_dmas = []
        for i, (off, rows) in enumerate(z_regions + rest_regions):
            d = pltpu.make_async_copy(
                x_ref.at[pl.ds(off, rows), :],
                stage.at[pl.ds(off, rows), :],
                insem.at[i],
            )
            d.start()
            in_dmas.append(d)

        barrier_sem = pltpu.get_barrier_semaphore()
        for nbr in (znbr, xnbr, ynbr):
            pl.semaphore_signal(
                barrier_sem, inc=1, device_id=nbr,
                device_id_type=pl.DeviceIdType.MESH,
            )
        pl.semaphore_wait(barrier_sem, 3)

        def rdma(off, rows, ssem, rsem, nbr, from_mine=False):
            r_ = pltpu.make_async_remote_copy(
                src_ref=(mine if from_mine else comm).at[pl.ds(off, rows), :],
                dst_ref=comm.at[pl.ds(off, rows), :],
                send_sem=ssem,
                recv_sem=rsem,
                device_id=nbr,
                device_id_type=pl.DeviceIdType.MESH,
            )
            r_.start()
            return r_

        copies = []

        def copy_out(off, rows, sem):
            cp = pltpu.make_async_copy(
                comm.at[pl.ds(off, rows), :],
                out_ref.at[pl.ds(other_base + off, rows), :],
                sem,
            )
            cp.start()
            copies.append(cp)

        x_in = chunk_offs(p_xn, PC) + chunk_offs(r_oth, RC[:2]) + \
            chunk_offs(p_dg, PC[:2])
        y_in = chunk_offs(p_yn, PC) + chunk_offs(r_oth + RH, RC[2:]) + \
            chunk_offs(p_dg + PH, PC[2:])

        def wait_in(slots, offs, rsem, lsem):
            for i in slots:
                off, rows = offs[i]
                w = pltpu.make_async_remote_copy(
                    src_ref=comm.at[pl.ds(off, rows), :],
                    dst_ref=comm.at[pl.ds(off, rows), :],
                    send_sem=zsend.at[0],
                    recv_sem=rsem.at[i],
                    device_id=znbr,
                    device_id_type=pl.DeviceIdType.MESH,
                )
                w.wait_recv()
                copy_out(off, rows, lsem.at[i])

        z_rdmas = []
        for i, (off, rows) in enumerate(z_regions):
            in_dmas[i].wait()
            mine[pl.ds(off, rows), :] = stage[pl.ds(off, rows), :].astype(
                jnp.bfloat16
            )
            z_rdmas.append(
                rdma(off, rows, zsend.at[i], zrecv.at[i], znbr, from_mine=True)
            )

        for i, (off, rows) in enumerate(rest_regions):
            in_dmas[8 + i].wait()
            mine[pl.ds(off, rows), :] = stage[pl.ds(off, rows), :].astype(
                jnp.bfloat16
            )
        mcopy = pltpu.make_async_copy(
            mine, out_ref.at[pl.ds(my_base, m), :], lmsem
        )
        mcopy.start()
        copies.append(mcopy)

        sends = []

        for j in range(4):
            z_rdmas[j].wait_recv()
            off, rows = z_regions[j]
            sends.append(rdma(off, rows, xsend.at[j], xrecv.at[j], xnbr))
            sends.append(rdma(off, rows, ysend.at[j], yrecv.at[j], ynbr))
            copy_out(off, rows, lzsem.at[j])
            if j == 1:
                wait_in([0], y_in, yrecv, lysem)
                sends.append(
                    rdma(p_yn, PC[0], xsend.at[6], xrecv.at[6], xnbr)
                )

        wait_in([1], y_in, yrecv, lysem)
        sends.append(
            rdma(p_yn + PC[0], PC[1], xsend.at[7], xrecv.at[7], xnbr)
        )

        wait_in([2], x_in, xrecv, lxsem)
        sends.append(
            rdma(p_xn + PH, PC[2], ysend.at[6], yrecv.at[6], ynbr)
        )
        wait_in([3], x_in, xrecv, lxsem)
        sends.append(
            rdma(p_xn + PH + PC[2], PC[3], ysend.at[7], yrecv.at[7], ynbr)
        )

        for j in range(4):
            z_rdmas[4 + j].wait_recv()
            off, rows = z_regions[4 + j]
            ssem, rsem, nbr = (
                (xsend, xrecv, xnbr) if j < 2 else (ysend, yrecv, ynbr)
            )
            sends.append(rdma(off, rows, ssem.at[4 + j % 2], rsem.at[4 + j % 2], nbr))
            copy_out(off, rows, lzsem.at[4 + j])

        wait_in([0, 1, 4, 5, 6, 7], x_in, xrecv, lxsem)
        wait_in([2, 3, 4, 5, 6, 7], y_in, yrecv, lysem)

        for r_ in z_rdmas:
            r_.wait_send()
        for r_ in sends:
            r_.wait_send()
        for cp in copies:
            cp.wait()

    return pl.pallas_call(
        body,
        out_shape=jax.ShapeDtypeStruct((2 * m, n), jnp.bfloat16),
        in_specs=[pl.BlockSpec(memory_space=pl.ANY)],
        out_specs=pl.BlockSpec(memory_space=pl.ANY),
        scratch_shapes=[
            pltpu.VMEM((m, n), jnp.float32),
            pltpu.VMEM((m, n), jnp.bfloat16),
            pltpu.VMEM((m, n), jnp.bfloat16),
            pltpu.SemaphoreType.DMA((12,)),
            pltpu.SemaphoreType.DMA((8,)),
            pltpu.SemaphoreType.DMA((8,)),
            pltpu.SemaphoreType.DMA((8,)),
            pltpu.SemaphoreType.DMA((8,)),
            pltpu.SemaphoreType.DMA((8,)),
            pltpu.SemaphoreType.DMA((8,)),
            pltpu.SemaphoreType.DMA((8,)),
            pltpu.SemaphoreType.DMA((8,)),
            pltpu.SemaphoreType.DMA((8,)),
            pltpu.SemaphoreType.DMA,
        ],
        compiler_params=pltpu.CompilerParams(collective_id=0),
    )(x)


# device time: 50083 ns/iter; 2.1170x vs baseline; 1.0015x over previous
import jax
import jax.numpy as jnp
from jax import lax
from jax.experimental import pallas as pl
from jax.experimental.pallas import tpu as pltpu

P = 672
PH = 336
R = 704
RH = 352

PC = (112, 224, 168, 168)
RC = (176, 176, 176, 176)


def kernel(x):
    m, n = x.shape
    assert 4 * P + 2 * R == m

    def body(
        x_ref, out_ref, stage, mine, comm,
        insem, zsend, zrecv, xsend, xrecv, ysend, yrecv,
        lzsem, lxsem, lysem, lmsem,
    ):
        my_x = lax.axis_index("x")
        my_y = lax.axis_index("y")
        my_z = lax.axis_index("z")
        znbr = (my_x, my_y, 1 - my_z)
        xnbr = (1 - my_x, my_y, my_z)
        ynbr = (my_x, 1 - my_y, my_z)
        c = (my_x + my_y) % 2
        my_base = my_z * m
        other_base = (1 - my_z) * m

        p_own = P * (my_x + 2 * my_y)
        p_xn = P * ((1 - my_x) + 2 * my_y)
        p_yn = P * (my_x + 2 * (1 - my_y))
        p_dg = P * ((1 - my_x) + 2 * (1 - my_y))
        r_own = 4 * P + R * c
        r_oth = 4 * P + R * (1 - c)

        def chunk_offs(base, sizes):
            offs, o = [], base
            for s in sizes:
                offs.append((o, s))
                o = o + s
            return offs

        z_regions = chunk_offs(p_own, PC) + chunk_offs(r_own, RC)
        rest_regions = [(p_xn, P), (p_yn, P), (p_dg, P), (r_oth, R)]

        in_dmas = []
        for i, (off, rows) in enumerate(z_regions + rest_regions):
            d = pltpu.make_async_copy(
                x_ref.at[pl.ds(off, rows), :],
                stage.at[pl.ds(off, rows), :],
                insem.at[i],
            )
            d.start()
            in_dmas.append(d)

        barrier_sem = pltpu.get_barrier_semaphore()
        for nbr in (znbr, xnbr, ynbr):
            pl.semaphore_signal(
                barrier_sem, inc=1, device_id=nbr,
                device_id_type=pl.DeviceIdType.MESH,
            )
        pl.semaphore_wait(barrier_sem, 3)

        def rdma(off, rows, ssem, rsem, nbr, from_mine=False):
            r_ = pltpu.make_async_remote_copy(
                src_ref=(mine if from_mine else comm).at[pl.ds(off, rows), :],
                dst_ref=comm.at[pl.ds(off, rows), :],
                send_sem=ssem,
                recv_sem=rsem,
                device_id=nbr,
                device_id_type=pl.DeviceIdType.MESH,
            )
            r_.start()
            return r_

        copies = []

        def copy_out(off, rows, sem):
            cp = pltpu.make_async_copy(
                comm.at[pl.ds(off, rows), :],
                out_ref.at[pl.ds(other_base + off, rows), :],
                sem,
            )
            cp.start()
            copies.append(cp)

        x_in = chunk_offs(p_xn, PC) + chunk_offs(r_oth, RC[:2]) + \
            chunk_offs(p_dg, PC[:2])
        y_in = chunk_offs(p_yn, PC) + chunk_offs(r_oth + RH, RC[2:]) + \
            chunk_offs(p_dg + PH, PC[2:])

        def wait_in(slots, offs, rsem, lsem):
            for i in slots:
                off, rows = offs[i]
                w = pltpu.make_async_remote_copy(
                    src_ref=comm.at[pl.ds(off, rows), :],
                    dst_ref=comm.at[pl.ds(off, rows), :],
                    send_sem=zsend.at[0],
                    recv_sem=rsem.at[i],
                    device_id=znbr,
                    device_id_type=pl.DeviceIdType.MESH,
                )
                w.wait_recv()
                copy_out(off, rows, lsem.at[i])

        z_rdmas = []
        for i, (off, rows) in enumerate(z_regions):
            in_dmas[i].wait()
            mine[pl.ds(off, rows), :] = stage[pl.ds(off, rows), :].astype(
                jnp.bfloat16
            )
            z_rdmas.append(
                rdma(off, rows, zsend.at[i], zrecv.at[i], znbr, from_mine=True)
            )

        sends = []

        for j in range(4):
            z_rdmas[j].wait_recv()
            off, rows = z_regions[j]
            sends.append(rdma(off, rows, xsend.at[j], xrecv.at[j], xnbr))
            sends.append(rdma(off, rows, ysend.at[j], yrecv.at[j], ynbr))
            copy_out(off, rows, lzsem.at[j])
            if j == 1:
                wait_in([0], y_in, yrecv, lysem)
                sends.append(
                    rdma(p_yn, PC[0], xsend.at[6], xrecv.at[6], xnbr)
                )

        wait_in([1], y_in, yrecv, lysem)
        sends.append(
            rdma(p_yn + PC[0], PC[1], xsend.at[7], xrecv.at[7], xnbr)
        )

        wait_in([2], x_in, xrecv, lxsem)
        sends.append(
            rdma(p_xn + PH, PC[2], ysend.at[6], yrecv.at[6], ynbr)
        )
        wait_in([3], x_in, xrecv, lxsem)
        sends.append(
            rdma(p_xn + PH + PC[2], PC[3], ysend.at[7], yrecv.at[7], ynbr)
        )

        for j in range(4):
            z_rdmas[4 + j].wait_recv()
            off, rows = z_regions[4 + j]
            ssem, rsem, nbr = (
                (xsend, xrecv, xnbr) if j < 2 else (ysend, yrecv, ynbr)
            )
            sends.append(rdma(off, rows, ssem.at[4 + j % 2], rsem.at[4 + j % 2], nbr))
            copy_out(off, rows, lzsem.at[4 + j])

        for i, (off, rows) in enumerate(rest_regions):
            in_dmas[8 + i].wait()
            mine[pl.ds(off, rows), :] = stage[pl.ds(off, rows), :].astype(
                jnp.bfloat16
            )
        mcopy = pltpu.make_async_copy(
            mine, out_ref.at[pl.ds(my_base, m), :], lmsem
        )
        mcopy.start()
        copies.append(mcopy)

        wait_in([0, 1, 4, 5, 6, 7], x_in, xrecv, lxsem)
        wait_in([2, 3, 4, 5, 6, 7], y_in, yrecv, lysem)

        for r_ in z_rdmas:
            r_.wait_send()
        for r_ in sends:
            r_.wait_send()
        for cp in copies:
            cp.wait()

    return pl.pallas_call(
        body,
        out_shape=jax.ShapeDtypeStruct((2 * m, n), jnp.bfloat16),
        in_specs=[pl.BlockSpec(memory_space=pl.ANY)],
        out_specs=pl.BlockSpec(memory_space=pl.ANY),
        scratch_shapes=[
            pltpu.VMEM((m, n), jnp.float32),
            pltpu.VMEM((m, n), jnp.bfloat16),
            pltpu.VMEM((m, n), jnp.bfloat16),
            pltpu.SemaphoreType.DMA((12,)),
            pltpu.SemaphoreType.DMA((8,)),
            pltpu.SemaphoreType.DMA((8,)),
            pltpu.SemaphoreType.DMA((8,)),
            pltpu.SemaphoreType.DMA((8,)),
            pltpu.SemaphoreType.DMA((8,)),
            pltpu.SemaphoreType.DMA((8,)),
            pltpu.SemaphoreType.DMA((8,)),
            pltpu.SemaphoreType.DMA((8,)),
            pltpu.SemaphoreType.DMA((8,)),
            pltpu.SemaphoreType.DMA,
        ],
        compiler_params=pltpu.CompilerParams(collective_id=0),
    )(x)
